# baseline (speedup 1.0000x reference)
import jax
import jax.numpy as jnp
from jax.experimental import pallas as pl
from jax.experimental.pallas import tpu as pltpu

BF16 = jnp.bfloat16
F32 = jnp.float32
LP = 10


def _dc1_body(n_in, *args):
    x_refs = args[:n_in]
    w1_refs = args[n_in:2 * n_in]
    b1_ref, w2_ref, b2_ref, o_ref = args[2 * n_in:]
    acc = jnp.dot(x_refs[0][...], w1_refs[0][...], preferred_element_type=F32)
    for i in range(1, n_in):
        acc = acc + jnp.dot(x_refs[i][...], w1_refs[i][...],
                            preferred_element_type=F32)
    h = jnp.maximum(acc + b1_ref[0], 0.0).astype(BF16)
    y = jnp.dot(h, w2_ref[...], preferred_element_type=F32)
    o_ref[...] = jnp.maximum(y + b2_ref[0], 0.0).astype(o_ref.dtype)


def _dc1(xs, w1, b1, w2, b2, tm=2048):
    B, H, W = xs[0].shape[:3]
    M = B * H * W
    x2 = [x.reshape(M, x.shape[-1]).astype(BF16) for x in xs]
    slabs, off = [], 0
    for x in x2:
        slabs.append(w1[off:off + x.shape[-1]])
        off += x.shape[-1]
    cmid, cout = w2.shape
    n_in = len(x2)

    in_specs = [pl.BlockSpec((tm, x.shape[-1]), lambda g: (g, 0)) for x in x2]
    in_specs += [pl.BlockSpec(w.shape, lambda g: (0, 0)) for w in slabs]
    in_specs += [pl.BlockSpec((1, cmid), lambda g: (0, 0)),
                 pl.BlockSpec((cmid, cout), lambda g: (0, 0)),
                 pl.BlockSpec((1, cout), lambda g: (0, 0))]

    out = pl.pallas_call(
        lambda *a: _dc1_body(n_in, *a),
        out_shape=jax.ShapeDtypeStruct((M, cout), BF16),
        grid=(M // tm,),
        in_specs=in_specs,
        out_specs=pl.BlockSpec((tm, cout), lambda g: (g, 0)),
        compiler_params=pltpu.CompilerParams(
            dimension_semantics=("parallel",)),
    )(*x2, *slabs, b1, w2, b2)
    return out.reshape(B, H, W, cout)


def _dc3_body(TH, H, W, Wp, cin, cmid, cout,
              x_hbm, w1_ref, b1_ref, w2_ref, b2_ref, o_ref, xbuf, sem):
    b = pl.program_id(0)
    t = pl.program_id(1)
    r0 = t * TH
    cp = pltpu.make_async_copy(x_hbm.at[b, pl.ds(r0, TH + 5)], xbuf, sem)
    cp.start()
    cp.wait()

    xflat = xbuf[...].reshape((TH + 5) * Wp, cin)
    M1 = (TH + 2) * Wp + 8
    M2 = TH * Wp

    acc = jnp.dot(xflat[0:M1], w1_ref[0], preferred_element_type=F32)
    for i in range(3):
        for j in range(3):
            if i == 0 and j == 0:
                continue
            o = i * Wp + j
            acc = acc + jnp.dot(xflat[o:o + M1], w1_ref[3 * i + j],
                                preferred_element_type=F32)
    hx = jnp.maximum(acc + b1_ref[0], 0.0)

    iota = jax.lax.broadcasted_iota(jnp.int32, (M1, cmid), 0)
    ic = iota % Wp + (1 - LP)
    ir = iota // Wp + (r0 - 1)
    valid = (ic >= 0) & (ic < W) & (ir >= 0) & (ir < H)
    h = jnp.where(valid, hx, 0.0).astype(BF16)

    acc2 = jnp.dot(h[0:M2], w2_ref[0], preferred_element_type=F32)
    for i in range(3):
        for j in range(3):
            if i == 0 and j == 0:
                continue
            o = i * Wp + j
            acc2 = acc2 + jnp.dot(h[o:o + M2], w2_ref[3 * i + j],
                                  preferred_element_type=F32)
    y = jnp.maximum(acc2 + b2_ref[0], 0.0)
    y3 = y.reshape(TH, Wp, cout)
    o_ref[0] = y3[:, LP - 2:LP - 2 + W, :].astype(o_ref.dtype)


def _dc3(x, w1, b1, w2, b2, th):
    B, H, W, cin = x.shape
    cmid, cout = w2.shape[1], w2.shape[2]
    Wp = W + 16
    nt = H // th
    xp = jnp.pad(x.astype(BF16),
                 ((0, 0), (2, 3), (LP, Wp - W - LP), (0, 0)))

    body = lambda *a: _dc3_body(th, H, W, Wp, cin, cmid, cout, *a)
    out = pl.pallas_call(
        body,
        out_shape=jax.ShapeDtypeStruct((B, H, W, cout), BF16),
        grid=(B, nt),
        in_specs=[
            pl.BlockSpec(memory_space=pl.ANY),
            pl.BlockSpec((9, cin, cmid), lambda b, t: (0, 0, 0)),
            pl.BlockSpec((1, cmid), lambda b, t: (0, 0)),
            pl.BlockSpec((9, cmid, cout), lambda b, t: (0, 0, 0)),
            pl.BlockSpec((1, cout), lambda b, t: (0, 0)),
        ],
        out_specs=pl.BlockSpec((1, th, W, cout), lambda b, t: (b, t, 0, 0)),
        scratch_shapes=[
            pltpu.VMEM((th + 5, Wp, cin), BF16),
            pltpu.SemaphoreType.DMA(()),
        ],
        compiler_params=pltpu.CompilerParams(
            dimension_semantics=("parallel", "parallel")),
    )(xp, w1, b1, w2, b2)
    return out


def _cls_body(a_ref, b_ref, wa_ref, wb_ref, bias_ref, o_ref):
    y = jnp.dot(a_ref[...], wa_ref[...], preferred_element_type=F32)
    y = y + jnp.dot(b_ref[...], wb_ref[...], preferred_element_type=F32)
    y = y + bias_ref[0]
    o_ref[...] = 1.0 / (1.0 + jnp.exp(-y))


def _classifier(xa, xb, w, bias, tm=2048):
    B, H, W = xa.shape[:3]
    M = B * H * W
    ca, cb = xa.shape[-1], xb.shape[-1]
    ncls = w.shape[1]
    a2 = xa.reshape(M, ca).astype(BF16)
    b2 = xb.reshape(M, cb).astype(BF16)

    out = pl.pallas_call(
        _cls_body,
        out_shape=jax.ShapeDtypeStruct((M, ncls), F32),
        grid=(M // tm,),
        in_specs=[
            pl.BlockSpec((tm, ca), lambda g: (g, 0)),
            pl.BlockSpec((tm, cb), lambda g: (g, 0)),
            pl.BlockSpec((ca, ncls), lambda g: (0, 0)),
            pl.BlockSpec((cb, ncls), lambda g: (0, 0)),
            pl.BlockSpec((1, ncls), lambda g: (0, 0)),
        ],
        out_specs=pl.BlockSpec((tm, ncls), lambda g: (g, 0)),
        compiler_params=pltpu.CompilerParams(
            dimension_semantics=("parallel",)),
    )(a2, b2, w[:ca], w[ca:], bias)
    return out.reshape(B, H, W, ncls)


def _maxpool(x, s):
    B, H, W, C = x.shape
    return x.reshape(B, H // s, s, W // s, s, C).max(axis=(2, 4))


def _phase_weights(s):
    ph = []
    for m in range(s):
        t = (m + 0.5) / s - 0.5
        if t < 0:
            ph.append((0, -t))
        else:
            ph.append((1, 1.0 - t))
    return ph


def _up_rows(x, s):
    B, H, W, C = x.shape
    xp = jnp.pad(x, ((0, 0), (1, 1), (0, 0), (0, 0)), mode="edge")
    phases = [wa * xp[:, a:a + H] + (1.0 - wa) * xp[:, a + 1:a + 1 + H]
              for a, wa in _phase_weights(s)]
    y = jnp.stack(phases, axis=2)
    return y.reshape(B, H * s, W, C)


def _up_cols(x, s):
    B, H, W, C = x.shape
    xp = jnp.pad(x, ((0, 0), (0, 0), (1, 1), (0, 0)), mode="edge")
    phases = [wa * xp[:, :, a:a + W] + (1.0 - wa) * xp[:, :, a + 1:a + 1 + W]
              for a, wa in _phase_weights(s)]
    y = jnp.stack(phases, axis=3)
    return y.reshape(B, H, W * s, C)


def _upsample(x, s):
    y = _up_cols(_up_rows(x.astype(F32), s), s)
    return y.astype(x.dtype)


def _th_for(H):
    return {256: 8, 128: 16, 64: 32}.get(H, 8)


def kernel(x, spect0_w1, spect0_b1, spect0_w2, spect0_b2, space0_w1, space0_b1, space0_w2, space0_b2, spect1_w1, spect1_b1, spect1_w2, spect1_b2, space1_w1, space1_b1, space1_w2, space1_b2, spect2_w1, spect2_b1, spect2_w2, spect2_b2, space2_w1, space2_b1, space2_w2, space2_b2, spect3_w1, spect3_b1, spect3_w2, spect3_b2, space3_w1, space3_b1, space3_w2, space3_b2, spect4_w1, spect4_b1, spect4_w2, spect4_b2, space4_w1, space4_b1, space4_w2, space4_b2, cls_w, cls_b):
    spect = [
        (spect0_w1, spect0_b1, spect0_w2, spect0_b2),
        (spect1_w1, spect1_b1, spect1_w2, spect1_b2),
        (spect2_w1, spect2_b1, spect2_w2, spect2_b2),
        (spect3_w1, spect3_b1, spect3_w2, spect3_b2),
        (spect4_w1, spect4_b1, spect4_w2, spect4_b2),
    ]
    space = [
        (space0_w1, space0_b1, space0_w2, space0_b2),
        (space1_w1, space1_b1, space1_w2, space1_b2),
        (space2_w1, space2_b1, space2_w2, space2_b2),
        (space3_w1, space3_b1, space3_w2, space3_b2),
        (space4_w1, space4_b1, space4_w2, space4_b2),
    ]
    xh = jnp.transpose(x, (0, 2, 3, 1)).astype(BF16)
    cat = lambda a, b: jnp.concatenate([a, b], axis=-1)

    st = _dc1([xh], *spect[0])
    sp = _dc3(xh, *space[0], th=_th_for(xh.shape[1]))

    st_in = [st, sp]
    sp_in = cat(_maxpool(sp, 2), _maxpool(st, 2))
    st = _dc1(st_in, *spect[1])
    sp = _dc3(sp_in, *space[1], th=_th_for(sp_in.shape[1]))

    st_in = [st, _upsample(sp, 2)]
    sp_in = cat(_maxpool(sp, 2), _maxpool(st, 4))
    st = _dc1(st_in, *spect[2])
    sp = _dc3(sp_in, *space[2], th=_th_for(sp_in.shape[1]))

    st_in = [st, _upsample(sp, 4)]
    sp_in = cat(_upsample(sp, 2), _maxpool(st, 2))
    st = _dc1(st_in, *spect[3])
    sp = _dc3(sp_in, *space[3], th=_th_for(sp_in.shape[1]))

    st_in = [st, _upsample(sp, 2)]
    sp_in = cat(_upsample(sp, 2), st)
    st = _dc1(st_in, *spect[4])
    sp = _dc3(sp_in, *space[4], th=_th_for(sp_in.shape[1]))

    out = _classifier(st, sp, cls_w, cls_b)
    B, H, W, _ = out.shape
    return out.reshape(B, H, W)


# speedup vs baseline: 1.2192x; 1.2192x over previous
import jax
import jax.numpy as jnp
from jax.experimental import pallas as pl
from jax.experimental.pallas import tpu as pltpu

BF16 = jnp.bfloat16
F32 = jnp.float32
LP = 10


def _dc1_body(n_in, *args):
    x_refs = args[:n_in]
    w1_refs = args[n_in:2 * n_in]
    b1_ref, w2_ref, b2_ref, o_ref = args[2 * n_in:]
    acc = jnp.dot(x_refs[0][...], w1_refs[0][...], preferred_element_type=F32)
    for i in range(1, n_in):
        acc = acc + jnp.dot(x_refs[i][...], w1_refs[i][...],
                            preferred_element_type=F32)
    h = jnp.maximum(acc + b1_ref[0], 0.0).astype(BF16)
    y = jnp.dot(h, w2_ref[...], preferred_element_type=F32)
    o_ref[...] = jnp.maximum(y + b2_ref[0], 0.0).astype(o_ref.dtype)


def _dc1(xs, w1, b1, w2, b2, tm=2048):
    B, H, W = xs[0].shape[:3]
    M = B * H * W
    x2 = [x.reshape(M, x.shape[-1]).astype(BF16) for x in xs]
    slabs, off = [], 0
    for x in x2:
        slabs.append(w1[off:off + x.shape[-1]])
        off += x.shape[-1]
    cmid, cout = w2.shape
    n_in = len(x2)

    in_specs = [pl.BlockSpec((tm, x.shape[-1]), lambda g: (g, 0)) for x in x2]
    in_specs += [pl.BlockSpec(w.shape, lambda g: (0, 0)) for w in slabs]
    in_specs += [pl.BlockSpec((1, cmid), lambda g: (0, 0)),
                 pl.BlockSpec((cmid, cout), lambda g: (0, 0)),
                 pl.BlockSpec((1, cout), lambda g: (0, 0))]

    out = pl.pallas_call(
        lambda *a: _dc1_body(n_in, *a),
        out_shape=jax.ShapeDtypeStruct((M, cout), BF16),
        grid=(M // tm,),
        in_specs=in_specs,
        out_specs=pl.BlockSpec((tm, cout), lambda g: (g, 0)),
        compiler_params=pltpu.CompilerParams(
            dimension_semantics=("parallel",)),
    )(*x2, *slabs, b1, w2, b2)
    return out.reshape(B, H, W, cout)


def _dc3_body(TH, NT, H, W, Wp, cin, cmid, cout,
              x_hbm, w1_ref, b1_ref, w2_ref, b2_ref, cmask_ref,
              o_ref, xbuf, sems):
    b = pl.program_id(0)
    t = pl.program_id(1)
    M1 = (TH + 2) * Wp + 8
    M2 = TH * Wp

    def copy_for(tt, s):
        return pltpu.make_async_copy(
            x_hbm.at[b, pl.ds(tt * TH, TH + 5)], xbuf.at[s], sems.at[s])

    slot = jax.lax.rem(t, 2)

    @pl.when(t == 0)
    def _():
        copy_for(t, slot).start()

    @pl.when(t < NT - 1)
    def _():
        copy_for(t + 1, jax.lax.rem(t + 1, 2)).start()

    copy_for(t, slot).wait()

    xflat = xbuf[slot].reshape((TH + 5) * Wp, cin)

    acc = jnp.dot(xflat[0:M1], w1_ref[0], preferred_element_type=F32)
    for i in range(3):
        for j in range(3):
            if i == 0 and j == 0:
                continue
            o = i * Wp + j
            acc = acc + jnp.dot(xflat[o:o + M1], w1_ref[3 * i + j],
                                preferred_element_type=F32)
    hb = jnp.maximum(acc + b1_ref[0], 0.0).astype(BF16)

    iota = jax.lax.broadcasted_iota(jnp.int32, (M1, cmid), 0)
    lo = jnp.where(t == 0, Wp, 0)
    hi = (H - t * TH + 1) * Wp
    h = jnp.where((iota >= lo) & (iota < hi), hb, 0) * cmask_ref[...]

    acc2 = jnp.dot(h[0:M2], w2_ref[0], preferred_element_type=F32)
    for i in range(3):
        for j in range(3):
            if i == 0 and j == 0:
                continue
            o = i * Wp + j
            acc2 = acc2 + jnp.dot(h[o:o + M2], w2_ref[3 * i + j],
                                  preferred_element_type=F32)
    y = jnp.maximum(acc2 + b2_ref[0], 0.0)
    y3 = y.reshape(TH, Wp, cout)
    o_ref[0] = y3[:, LP - 2:LP - 2 + W, :].astype(o_ref.dtype)


def _dc3(x, w1, b1, w2, b2, th):
    B, H, W, cin = x.shape
    cmid, cout = w2.shape[1], w2.shape[2]
    Wp = W + 16
    nt = H // th
    xp = jnp.pad(x.astype(BF16),
                 ((0, 0), (2, 3), (LP, Wp - W - LP), (0, 0)))

    M1 = (th + 2) * Wp + 8
    pp = jnp.arange(M1) % Wp
    cm = ((pp >= LP - 1) & (pp <= LP + W - 2)).astype(BF16)
    cmask = jnp.broadcast_to(cm[:, None], (M1, cmid))

    body = lambda *a: _dc3_body(th, nt, H, W, Wp, cin, cmid, cout, *a)
    out = pl.pallas_call(
        body,
        out_shape=jax.ShapeDtypeStruct((B, H, W, cout), BF16),
        grid=(B, nt),
        in_specs=[
            pl.BlockSpec(memory_space=pl.ANY),
            pl.BlockSpec((9, cin, cmid), lambda b, t: (0, 0, 0)),
            pl.BlockSpec((1, cmid), lambda b, t: (0, 0)),
            pl.BlockSpec((9, cmid, cout), lambda b, t: (0, 0, 0)),
            pl.BlockSpec((1, cout), lambda b, t: (0, 0)),
            pl.BlockSpec((M1, cmid), lambda b, t: (0, 0)),
        ],
        out_specs=pl.BlockSpec((1, th, W, cout), lambda b, t: (b, t, 0, 0)),
        scratch_shapes=[
            pltpu.VMEM((2, th + 5, Wp, cin), BF16),
            pltpu.SemaphoreType.DMA((2,)),
        ],
        compiler_params=pltpu.CompilerParams(
            dimension_semantics=("parallel", "arbitrary")),
    )(xp, w1, b1, w2, b2, cmask)
    return out


def _cls_body(a_ref, b_ref, wa_ref, wb_ref, bias_ref, o_ref):
    y = jnp.dot(a_ref[...], wa_ref[...], preferred_element_type=F32)
    y = y + jnp.dot(b_ref[...], wb_ref[...], preferred_element_type=F32)
    y = y + bias_ref[0]
    o_ref[...] = 1.0 / (1.0 + jnp.exp(-y))


def _classifier(xa, xb, w, bias, tm=2048):
    B, H, W = xa.shape[:3]
    M = B * H * W
    ca, cb = xa.shape[-1], xb.shape[-1]
    ncls = w.shape[1]
    a2 = xa.reshape(M, ca).astype(BF16)
    b2 = xb.reshape(M, cb).astype(BF16)

    out = pl.pallas_call(
        _cls_body,
        out_shape=jax.ShapeDtypeStruct((M, ncls), F32),
        grid=(M // tm,),
        in_specs=[
            pl.BlockSpec((tm, ca), lambda g: (g, 0)),
            pl.BlockSpec((tm, cb), lambda g: (g, 0)),
            pl.BlockSpec((ca, ncls), lambda g: (0, 0)),
            pl.BlockSpec((cb, ncls), lambda g: (0, 0)),
            pl.BlockSpec((1, ncls), lambda g: (0, 0)),
        ],
        out_specs=pl.BlockSpec((tm, ncls), lambda g: (g, 0)),
        compiler_params=pltpu.CompilerParams(
            dimension_semantics=("parallel",)),
    )(a2, b2, w[:ca], w[ca:], bias)
    return out.reshape(B, H, W, ncls)


def _maxpool(x, s):
    B, H, W, C = x.shape
    return x.reshape(B, H // s, s, W // s, s, C).max(axis=(2, 4))


def _phase_weights(s):
    ph = []
    for m in range(s):
        t = (m + 0.5) / s - 0.5
        if t < 0:
            ph.append((0, -t))
        else:
            ph.append((1, 1.0 - t))
    return ph


def _up_rows(x, s):
    B, H, W, C = x.shape
    xp = jnp.pad(x, ((0, 0), (1, 1), (0, 0), (0, 0)), mode="edge")
    phases = [wa * xp[:, a:a + H] + (1.0 - wa) * xp[:, a + 1:a + 1 + H]
              for a, wa in _phase_weights(s)]
    y = jnp.stack(phases, axis=2)
    return y.reshape(B, H * s, W, C)


def _up_cols(x, s):
    B, H, W, C = x.shape
    xp = jnp.pad(x, ((0, 0), (0, 0), (1, 1), (0, 0)), mode="edge")
    phases = [wa * xp[:, :, a:a + W] + (1.0 - wa) * xp[:, :, a + 1:a + 1 + W]
              for a, wa in _phase_weights(s)]
    y = jnp.stack(phases, axis=3)
    return y.reshape(B, H, W * s, C)


def _upsample(x, s):
    B, H, W, C = x.shape
    y = jax.image.resize(x.astype(F32), (B, H * s, W * s, C),
                         method="bilinear")
    return y.astype(x.dtype)


def _th_for(H):
    return {256: 16, 128: 16, 64: 32}.get(H, 16)


def kernel(x, spect0_w1, spect0_b1, spect0_w2, spect0_b2, space0_w1, space0_b1, space0_w2, space0_b2, spect1_w1, spect1_b1, spect1_w2, spect1_b2, space1_w1, space1_b1, space1_w2, space1_b2, spect2_w1, spect2_b1, spect2_w2, spect2_b2, space2_w1, space2_b1, space2_w2, space2_b2, spect3_w1, spect3_b1, spect3_w2, spect3_b2, space3_w1, space3_b1, space3_w2, space3_b2, spect4_w1, spect4_b1, spect4_w2, spect4_b2, space4_w1, space4_b1, space4_w2, space4_b2, cls_w, cls_b):
    spect = [
        (spect0_w1, spect0_b1, spect0_w2, spect0_b2),
        (spect1_w1, spect1_b1, spect1_w2, spect1_b2),
        (spect2_w1, spect2_b1, spect2_w2, spect2_b2),
        (spect3_w1, spect3_b1, spect3_w2, spect3_b2),
        (spect4_w1, spect4_b1, spect4_w2, spect4_b2),
    ]
    space = [
        (space0_w1, space0_b1, space0_w2, space0_b2),
        (space1_w1, space1_b1, space1_w2, space1_b2),
        (space2_w1, space2_b1, space2_w2, space2_b2),
        (space3_w1, space3_b1, space3_w2, space3_b2),
        (space4_w1, space4_b1, space4_w2, space4_b2),
    ]
    xh = jnp.transpose(x, (0, 2, 3, 1)).astype(BF16)
    cat = lambda a, b: jnp.concatenate([a, b], axis=-1)

    st = _dc1([xh], *spect[0])
    sp = _dc3(xh, *space[0], th=_th_for(xh.shape[1]))

    st_in = [st, sp]
    sp_in = cat(_maxpool(sp, 2), _maxpool(st, 2))
    st = _dc1(st_in, *spect[1])
    sp = _dc3(sp_in, *space[1], th=_th_for(sp_in.shape[1]))

    st_in = [st, _upsample(sp, 2)]
    sp_in = cat(_maxpool(sp, 2), _maxpool(st, 4))
    st = _dc1(st_in, *spect[2])
    sp = _dc3(sp_in, *space[2], th=_th_for(sp_in.shape[1]))

    st_in = [st, _upsample(sp, 4)]
    sp_in = cat(_upsample(sp, 2), _maxpool(st, 2))
    st = _dc1(st_in, *spect[3])
    sp = _dc3(sp_in, *space[3], th=_th_for(sp_in.shape[1]))

    st_in = [st, _upsample(sp, 2)]
    sp_in = cat(_upsample(sp, 2), st)
    st = _dc1(st_in, *spect[4])
    sp = _dc3(sp_in, *space[4], th=_th_for(sp_in.shape[1]))

    out = _classifier(st, sp, cls_w, cls_b)
    B, H, W, _ = out.shape
    return out.reshape(B, H, W)


# speedup vs baseline: 1.6899x; 1.3861x over previous
import jax
import jax.numpy as jnp
from jax.experimental import pallas as pl
from jax.experimental.pallas import tpu as pltpu

BF16 = jnp.bfloat16
F32 = jnp.float32
LP = 10


def _dc1_body(n_in, *args):
    x_refs = args[:n_in]
    w1_refs = args[n_in:2 * n_in]
    b1_ref, w2_ref, b2_ref, o_ref = args[2 * n_in:]
    acc = jnp.dot(x_refs[0][...], w1_refs[0][...], preferred_element_type=F32)
    for i in range(1, n_in):
        acc = acc + jnp.dot(x_refs[i][...], w1_refs[i][...],
                            preferred_element_type=F32)
    h = jnp.maximum(acc + b1_ref[0], 0.0).astype(BF16)
    y = jnp.dot(h, w2_ref[...], preferred_element_type=F32)
    o_ref[...] = jnp.maximum(y + b2_ref[0], 0.0).astype(o_ref.dtype)


def _dc1(xs, w1, b1, w2, b2, tm=2048):
    B, H, W = xs[0].shape[:3]
    M = B * H * W
    x2 = [x.reshape(M, x.shape[-1]).astype(BF16) for x in xs]
    slabs, off = [], 0
    for x in x2:
        slabs.append(w1[off:off + x.shape[-1]])
        off += x.shape[-1]
    cmid, cout = w2.shape
    n_in = len(x2)

    in_specs = [pl.BlockSpec((tm, x.shape[-1]), lambda g: (g, 0)) for x in x2]
    in_specs += [pl.BlockSpec(w.shape, lambda g: (0, 0)) for w in slabs]
    in_specs += [pl.BlockSpec((1, cmid), lambda g: (0, 0)),
                 pl.BlockSpec((cmid, cout), lambda g: (0, 0)),
                 pl.BlockSpec((1, cout), lambda g: (0, 0))]

    out = pl.pallas_call(
        lambda *a: _dc1_body(n_in, *a),
        out_shape=jax.ShapeDtypeStruct((M, cout), BF16),
        grid=(M // tm,),
        in_specs=in_specs,
        out_specs=pl.BlockSpec((tm, cout), lambda g: (g, 0)),
        compiler_params=pltpu.CompilerParams(
            dimension_semantics=("parallel",)),
    )(*x2, *slabs, b1, w2, b2)
    return out.reshape(B, H, W, cout)


def _dc3_body(TH, NT, H, W, Wp, cin, cmid, cout,
              x_hbm, w1_ref, b1_ref, w2_ref, b2_ref, cmask_ref,
              o_ref, xbuf, sems):
    b = pl.program_id(0)
    t = pl.program_id(1)
    M1 = (TH + 2) * Wp + 8
    M2 = TH * Wp

    def copy_for(tt, s):
        return pltpu.make_async_copy(
            x_hbm.at[b, pl.ds(tt * TH, TH + 5)], xbuf.at[s], sems.at[s])

    slot = jax.lax.rem(t, 2)

    @pl.when(t == 0)
    def _():
        copy_for(t, slot).start()

    @pl.when(t < NT - 1)
    def _():
        copy_for(t + 1, jax.lax.rem(t + 1, 2)).start()

    copy_for(t, slot).wait()

    xflat = xbuf[slot].reshape((TH + 5) * Wp, cin)

    Mb1 = M1 + 2 * Wp
    q = jnp.dot(xflat[0:Mb1], w1_ref[0], preferred_element_type=F32)
    for j in (1, 2):
        q = q + jnp.dot(xflat[j:j + Mb1], w1_ref[j],
                        preferred_element_type=F32)
    acc = q[0:M1, 0:cmid]
    for i in (1, 2):
        acc = acc + q[i * Wp:i * Wp + M1, i * cmid:(i + 1) * cmid]
    hb = jnp.maximum(acc + b1_ref[0], 0.0).astype(BF16)

    iota = jax.lax.broadcasted_iota(jnp.int32, (M1, cmid), 0)
    lo = jnp.where(t == 0, Wp, 0)
    hi = (H - t * TH + 1) * Wp
    h = jnp.where((iota >= lo) & (iota < hi), hb, 0) * cmask_ref[...]

    Mb2 = M2 + 2 * Wp
    q2 = jnp.dot(h[0:Mb2], w2_ref[0], preferred_element_type=F32)
    for j in (1, 2):
        q2 = q2 + jnp.dot(h[j:j + Mb2], w2_ref[j],
                          preferred_element_type=F32)
    acc2 = q2[0:M2, 0:cout]
    for i in (1, 2):
        acc2 = acc2 + q2[i * Wp:i * Wp + M2, i * cout:(i + 1) * cout]
    y = jnp.maximum(acc2 + b2_ref[0], 0.0)
    y3 = y.reshape(TH, Wp, cout)
    o_ref[0] = y3[:, LP - 2:LP - 2 + W, :].astype(o_ref.dtype)


def _dc3(x, w1, b1, w2, b2, th):
    B, H, W, cin = x.shape
    cmid, cout = w2.shape[1], w2.shape[2]
    Wp = W + 16
    nt = H // th
    xp = jnp.pad(x.astype(BF16),
                 ((0, 0), (2, 3), (LP, Wp - W - LP), (0, 0)))

    M1 = (th + 2) * Wp + 8
    pp = jnp.arange(M1) % Wp
    cm = ((pp >= LP - 1) & (pp <= LP + W - 2)).astype(BF16)
    cmask = jnp.broadcast_to(cm[:, None], (M1, cmid))

    w1p = jnp.stack([jnp.concatenate([w1[j], w1[3 + j], w1[6 + j]], axis=-1)
                     for j in range(3)])
    w2p = jnp.stack([jnp.concatenate([w2[j], w2[3 + j], w2[6 + j]], axis=-1)
                     for j in range(3)])

    body = lambda *a: _dc3_body(th, nt, H, W, Wp, cin, cmid, cout, *a)
    out = pl.pallas_call(
        body,
        out_shape=jax.ShapeDtypeStruct((B, H, W, cout), BF16),
        grid=(B, nt),
        in_specs=[
            pl.BlockSpec(memory_space=pl.ANY),
            pl.BlockSpec((3, cin, 3 * cmid), lambda b, t: (0, 0, 0)),
            pl.BlockSpec((1, cmid), lambda b, t: (0, 0)),
            pl.BlockSpec((3, cmid, 3 * cout), lambda b, t: (0, 0, 0)),
            pl.BlockSpec((1, cout), lambda b, t: (0, 0)),
            pl.BlockSpec((M1, cmid), lambda b, t: (0, 0)),
        ],
        out_specs=pl.BlockSpec((1, th, W, cout), lambda b, t: (b, t, 0, 0)),
        scratch_shapes=[
            pltpu.VMEM((2, th + 5, Wp, cin), BF16),
            pltpu.SemaphoreType.DMA((2,)),
        ],
        compiler_params=pltpu.CompilerParams(
            dimension_semantics=("parallel", "arbitrary")),
    )(xp, w1p, b1, w2p, b2, cmask)
    return out


def _cls_body(a_ref, b_ref, wa_ref, wb_ref, bias_ref, o_ref):
    y = jnp.dot(a_ref[...], wa_ref[...], preferred_element_type=F32)
    y = y + jnp.dot(b_ref[...], wb_ref[...], preferred_element_type=F32)
    y = y + bias_ref[0]
    o_ref[...] = 1.0 / (1.0 + jnp.exp(-y))


def _classifier(xa, xb, w, bias, tm=2048):
    B, H, W = xa.shape[:3]
    M = B * H * W
    ca, cb = xa.shape[-1], xb.shape[-1]
    ncls = w.shape[1]
    a2 = xa.reshape(M, ca).astype(BF16)
    b2 = xb.reshape(M, cb).astype(BF16)

    out = pl.pallas_call(
        _cls_body,
        out_shape=jax.ShapeDtypeStruct((M, ncls), F32),
        grid=(M // tm,),
        in_specs=[
            pl.BlockSpec((tm, ca), lambda g: (g, 0)),
            pl.BlockSpec((tm, cb), lambda g: (g, 0)),
            pl.BlockSpec((ca, ncls), lambda g: (0, 0)),
            pl.BlockSpec((cb, ncls), lambda g: (0, 0)),
            pl.BlockSpec((1, ncls), lambda g: (0, 0)),
        ],
        out_specs=pl.BlockSpec((tm, ncls), lambda g: (g, 0)),
        compiler_params=pltpu.CompilerParams(
            dimension_semantics=("parallel",)),
    )(a2, b2, w[:ca], w[ca:], bias)
    return out.reshape(B, H, W, ncls)


def _maxpool(x, s):
    B, H, W, C = x.shape
    return x.reshape(B, H // s, s, W // s, s, C).max(axis=(2, 4))


def _phase_weights(s):
    ph = []
    for m in range(s):
        t = (m + 0.5) / s - 0.5
        if t < 0:
            ph.append((0, -t))
        else:
            ph.append((1, 1.0 - t))
    return ph


def _up_rows(x, s):
    B, H, W, C = x.shape
    xp = jnp.pad(x, ((0, 0), (1, 1), (0, 0), (0, 0)), mode="edge")
    phases = [wa * xp[:, a:a + H] + (1.0 - wa) * xp[:, a + 1:a + 1 + H]
              for a, wa in _phase_weights(s)]
    y = jnp.stack(phases, axis=2)
    return y.reshape(B, H * s, W, C)


def _up_cols(x, s):
    B, H, W, C = x.shape
    xp = jnp.pad(x, ((0, 0), (0, 0), (1, 1), (0, 0)), mode="edge")
    phases = [wa * xp[:, :, a:a + W] + (1.0 - wa) * xp[:, :, a + 1:a + 1 + W]
              for a, wa in _phase_weights(s)]
    y = jnp.stack(phases, axis=3)
    return y.reshape(B, H, W * s, C)


def _upsample(x, s):
    B, H, W, C = x.shape
    y = jax.image.resize(x.astype(F32), (B, H * s, W * s, C),
                         method="bilinear")
    return y.astype(x.dtype)


def _th_for(H):
    return {256: 16, 128: 16, 64: 32}.get(H, 16)


def kernel(x, spect0_w1, spect0_b1, spect0_w2, spect0_b2, space0_w1, space0_b1, space0_w2, space0_b2, spect1_w1, spect1_b1, spect1_w2, spect1_b2, space1_w1, space1_b1, space1_w2, space1_b2, spect2_w1, spect2_b1, spect2_w2, spect2_b2, space2_w1, space2_b1, space2_w2, space2_b2, spect3_w1, spect3_b1, spect3_w2, spect3_b2, space3_w1, space3_b1, space3_w2, space3_b2, spect4_w1, spect4_b1, spect4_w2, spect4_b2, space4_w1, space4_b1, space4_w2, space4_b2, cls_w, cls_b):
    spect = [
        (spect0_w1, spect0_b1, spect0_w2, spect0_b2),
        (spect1_w1, spect1_b1, spect1_w2, spect1_b2),
        (spect2_w1, spect2_b1, spect2_w2, spect2_b2),
        (spect3_w1, spect3_b1, spect3_w2, spect3_b2),
        (spect4_w1, spect4_b1, spect4_w2, spect4_b2),
    ]
    space = [
        (space0_w1, space0_b1, space0_w2, space0_b2),
        (space1_w1, space1_b1, space1_w2, space1_b2),
        (space2_w1, space2_b1, space2_w2, space2_b2),
        (space3_w1, space3_b1, space3_w2, space3_b2),
        (space4_w1, space4_b1, space4_w2, space4_b2),
    ]
    xh = jnp.transpose(x, (0, 2, 3, 1)).astype(BF16)
    cat = lambda a, b: jnp.concatenate([a, b], axis=-1)

    st = _dc1([xh], *spect[0])
    sp = _dc3(xh, *space[0], th=_th_for(xh.shape[1]))

    st_in = [st, sp]
    sp_in = cat(_maxpool(sp, 2), _maxpool(st, 2))
    st = _dc1(st_in, *spect[1])
    sp = _dc3(sp_in, *space[1], th=_th_for(sp_in.shape[1]))

    st_in = [st, _upsample(sp, 2)]
    sp_in = cat(_maxpool(sp, 2), _maxpool(st, 4))
    st = _dc1(st_in, *spect[2])
    sp = _dc3(sp_in, *space[2], th=_th_for(sp_in.shape[1]))

    st_in = [st, _upsample(sp, 4)]
    sp_in = cat(_upsample(sp, 2), _maxpool(st, 2))
    st = _dc1(st_in, *spect[3])
    sp = _dc3(sp_in, *space[3], th=_th_for(sp_in.shape[1]))

    st_in = [st, _upsample(sp, 2)]
    sp_in = cat(_upsample(sp, 2), st)
    st = _dc1(st_in, *spect[4])
    sp = _dc3(sp_in, *space[4], th=_th_for(sp_in.shape[1]))

    out = _classifier(st, sp, cls_w, cls_b)
    B, H, W, _ = out.shape
    return out.reshape(B, H, W)
